# baseline (speedup 1.0000x reference)
import functools

import jax
import jax.numpy as jnp
from jax import lax
from jax.experimental import pallas as pl
from jax.experimental.pallas import tpu as pltpu
from jax.experimental.pallas import tpu_sc as plsc

NC = 2
NS = 16
NW = NC * NS
LANES = 16

BATCH = 16384
DIM = 128
B_PER_W = BATCH // NW
CHUNK = 128
N_CHUNKS = B_PER_W // CHUNK
N_SLICES = DIM // LANES


def _sc_body(x_hbm, idx_hbm, table_hbm, icept_hbm, out_hbm,
             idx_v, x_v, rows_v, bias_v, out_v, tr_v,
             sem_t, sem_b, sem_x):
    wid = lax.axis_index("s") * NC + lax.axis_index("c")
    lane_iota = lax.iota(jnp.int32, LANES)
    wbase = wid * B_PER_W

    def issue(c):
        buf = c % 2
        base = wbase + c * CHUNK
        pltpu.sync_copy(idx_hbm.at[pl.ds(base, CHUNK)], idx_v[buf])
        row_cp = pltpu.async_copy(table_hbm.at[idx_v[buf]], rows_v[buf],
                                  sem_t[buf])
        bias_cp = pltpu.async_copy(icept_hbm.at[idx_v[buf]], bias_v[buf],
                                   sem_b[buf])
        x_cp = pltpu.async_copy(x_hbm.at[pl.ds(base, CHUNK)], x_v[buf],
                                sem_x[buf])
        return row_cp, bias_cp, x_cp

    def compute(c):
        buf = c % 2
        xb, rb, bb = x_v[buf], rows_v[buf], bias_v[buf]
        base = wbase + c * CHUNK

        def group_step(g, _):
            gbase = g * LANES
            for j in range(LANES):
                r = gbase + j
                p = [xb[r, pl.ds(di * LANES, LANES)] *
                     rb[r, pl.ds(di * LANES, LANES)]
                     for di in range(N_SLICES)]
                acc = ((p[0] + p[1]) + (p[2] + p[3]) +
                       ((p[4] + p[5]) + (p[6] + p[7])))
                tr_v[j, pl.ds(0, LANES)] = acc
            outacc = bb[pl.ds(gbase, LANES)]
            for cc in range(LANES):
                col = jnp.full((LANES,), cc, jnp.int32)
                outacc = outacc + plsc.load_gather(tr_v, [lane_iota, col])
            out_v[pl.ds(gbase, LANES)] = outacc
            return 0

        lax.fori_loop(0, CHUNK // LANES, group_step, 0)
        pltpu.sync_copy(out_v, out_hbm.at[pl.ds(base, CHUNK)])

    pend = issue(0)
    for c in range(N_CHUNKS):
        nxt = issue(c + 1) if c + 1 < N_CHUNKS else None
        for cp in pend:
            cp.wait()
        compute(c)
        pend = nxt


@jax.jit
def _alo_forward(x, index, table, icept):
    mesh = plsc.VectorSubcoreMesh(
        core_axis_name="c", subcore_axis_name="s",
        num_cores=NC, num_subcores=NS)
    dbl = lambda t: [t, t]
    run = pl.kernel(
        _sc_body,
        out_type=jax.ShapeDtypeStruct((BATCH,), jnp.float32),
        mesh=mesh,
        compiler_params=pltpu.CompilerParams(needs_layout_passes=False),
        scratch_types=[
            dbl(pltpu.VMEM((CHUNK,), jnp.int32)),
            dbl(pltpu.VMEM((CHUNK, DIM), jnp.float32)),
            dbl(pltpu.VMEM((CHUNK, DIM), jnp.float32)),
            dbl(pltpu.VMEM((CHUNK,), jnp.float32)),
            pltpu.VMEM((CHUNK,), jnp.float32),
            pltpu.VMEM((LANES, LANES + 1), jnp.float32),
            dbl(pltpu.SemaphoreType.DMA),
            dbl(pltpu.SemaphoreType.DMA),
            dbl(pltpu.SemaphoreType.DMA),
        ],
    )
    return run(x, index, table, icept)


def kernel(x, index, treeplus_loo_layer, treeplus_loo_intercept):
    index = index.astype(jnp.int32)
    return _alo_forward(x, index, treeplus_loo_layer, treeplus_loo_intercept)
